# baseline (speedup 1.0000x reference)
import functools

import jax
import jax.numpy as jnp
from jax import lax
from jax.experimental import pallas as pl
from jax.experimental.pallas import tpu as pltpu
from jax.experimental.pallas import tpu_sc as plsc

DIM = 64
BATCH = 16384
NUM_CORES = 2
NUM_SUBCORES = 16
NUM_WORKERS = NUM_CORES * NUM_SUBCORES
BPW = BATCH // NUM_WORKERS
CHUNK = 128
NCHUNK = BPW // CHUNK
CGROUPS = CHUNK // 16
_PACK_BW = 512


def _pack_half(n):
    return ((n // 2 + _PACK_BW - 1) // _PACK_BW) * _PACK_BW


UHALF = _pack_half(100000)
IHALF = _pack_half(1000000)

_mesh = plsc.VectorSubcoreMesh(core_axis_name="c", subcore_axis_name="s")


@functools.partial(
    pl.kernel,
    out_type=(
        jax.ShapeDtypeStruct((BATCH,), jnp.float32),
        jax.ShapeDtypeStruct((BATCH,), jnp.float32),
    ),
    mesh=_mesh,
    scratch_types=[
        pltpu.VMEM((NCHUNK, CHUNK), jnp.int32),
        pltpu.VMEM((NCHUNK, CHUNK), jnp.int32),
        pltpu.VMEM((NCHUNK, CHUNK), jnp.int32),
        pltpu.VMEM((NCHUNK, CHUNK), jnp.int32),
        pltpu.VMEM((NCHUNK, CHUNK), jnp.int32),
        pltpu.VMEM((NCHUNK, CHUNK), jnp.int32),
        pltpu.VMEM((CHUNK, 2 * DIM), jnp.float32),
        pltpu.VMEM((CHUNK, 2 * DIM), jnp.float32),
        pltpu.VMEM((CHUNK, 2 * DIM), jnp.float32),
        pltpu.VMEM((CHUNK, 2 * DIM), jnp.float32),
        pltpu.VMEM((CHUNK, 2 * DIM), jnp.float32),
        pltpu.VMEM((CHUNK, 2 * DIM), jnp.float32),
        pltpu.VMEM((BPW,), jnp.float32),
        pltpu.VMEM((BPW,), jnp.float32),
        pltpu.SemaphoreType.DMA,
    ],
    compiler_params=pltpu.CompilerParams(
        needs_layout_passes=False, use_tc_tiling_on_sc=True
    ),
)
def _bprmf_sc(u_hbm, pi_hbm, ni_hbm, user_t, item_t, pos_hbm, neg_hbm,
              u_v, pi_v, ni_v, uh_v, ph_v, nh_v,
              ue_v0, ue_v1, pe_v0, pe_v1, ne_v0, ne_v1,
              pos_v, neg_v, sem):
    wid = lax.axis_index("s") * NUM_CORES + lax.axis_index("c")
    lane = lax.iota(jnp.int32, 16)
    ue_b = (ue_v0, ue_v1)
    pe_b = (pe_v0, pe_v1)
    ne_b = (ne_v0, ne_v1)

    pltpu.sync_copy(u_hbm.at[pl.ds(wid * NCHUNK, NCHUNK)], u_v)
    pltpu.sync_copy(pi_hbm.at[pl.ds(wid * NCHUNK, NCHUNK)], pi_v)
    pltpu.sync_copy(ni_hbm.at[pl.ds(wid * NCHUNK, NCHUNK)], ni_v)

    for j in range(NCHUNK):
        for g in range(CGROUPS):
            s = pl.ds(g * 16, 16)
            uv, pv, nv = u_v[j, s], pi_v[j, s], ni_v[j, s]
            uh_v[j, s] = uv - jnp.where(uv >= UHALF, UHALF, 0)
            ph_v[j, s] = pv - jnp.where(pv >= IHALF, IHALF, 0)
            nh_v[j, s] = nv - jnp.where(nv >= IHALF, IHALF, 0)

    def fire(j, buf):
        return (
            pltpu.async_copy(user_t.at[uh_v.at[j]], ue_b[buf], sem),
            pltpu.async_copy(item_t.at[ph_v.at[j]], pe_b[buf], sem),
            pltpu.async_copy(item_t.at[nh_v.at[j]], ne_b[buf], sem),
        )

    def compute(j, buf):
        ue, pe, ne = ue_b[buf], pe_b[buf], ne_b[buf]

        def group_body(g, carry):
            s = pl.ds(g * 16, 16)
            ucol0 = jnp.where(u_v[j, s] >= UHALF, DIM, 0)
            pcol0 = jnp.where(pi_v[j, s] >= IHALF, DIM, 0)
            ncol0 = jnp.where(ni_v[j, s] >= IHALF, DIM, 0)
            rows = g * 16 + lane
            acc_p = jnp.zeros((16,), jnp.float32)
            acc_n = jnp.zeros((16,), jnp.float32)
            for d in range(DIM):
                rot = jnp.bitwise_and(lane + d, DIM - 1)
                a = plsc.load_gather(ue, [rows, ucol0 + rot])
                b = plsc.load_gather(pe, [rows, pcol0 + rot])
                c = plsc.load_gather(ne, [rows, ncol0 + rot])
                acc_p = acc_p + a * b
                acc_n = acc_n + a * c
            pos_v[pl.ds(j * CHUNK + g * 16, 16)] = acc_p
            neg_v[pl.ds(j * CHUNK + g * 16, 16)] = acc_n
            return carry

        lax.fori_loop(0, CGROUPS, group_body, 0)

    inflight = fire(0, 0)
    for j in range(NCHUNK):
        buf = j % 2
        for cp in inflight:
            cp.wait()
        if j + 1 < NCHUNK:
            inflight = fire(j + 1, 1 - buf)
        compute(j, buf)

    pltpu.sync_copy(pos_v, pos_hbm.at[pl.ds(wid * BPW, BPW)])
    pltpu.sync_copy(neg_v, neg_hbm.at[pl.ds(wid * BPW, BPW)])


def _pack_body(lo_ref, hi_ref, out_ref):
    lo = jnp.transpose(lo_ref[...])
    hi = jnp.transpose(hi_ref[...])
    out_ref[...] = jnp.concatenate([lo, hi], axis=1)


def _tc_pack(t_T):
    n = t_T.shape[1]
    half = _pack_half(n)
    half_blocks = half // _PACK_BW
    return pl.pallas_call(
        _pack_body,
        grid=(half_blocks,),
        in_specs=[
            pl.BlockSpec((DIM, _PACK_BW), lambda j: (0, j)),
            pl.BlockSpec((DIM, _PACK_BW), lambda j, hb=half_blocks: (0, j + hb)),
        ],
        out_specs=pl.BlockSpec((_PACK_BW, 2 * DIM), lambda j: (j, 0)),
        out_shape=jax.ShapeDtypeStruct((half, 2 * DIM), jnp.float32),
    )(t_T, t_T)


def kernel(u, pi, ni, user_emb, item_emb):
    u2 = u.astype(jnp.int32).reshape(BATCH // CHUNK, CHUNK)
    pi2 = pi.astype(jnp.int32).reshape(BATCH // CHUNK, CHUNK)
    ni2 = ni.astype(jnp.int32).reshape(BATCH // CHUNK, CHUNK)
    user2 = _tc_pack(user_emb.T)
    item2 = _tc_pack(item_emb.T)
    return _bprmf_sc(u2, pi2, ni2, user2, item2)


# speedup vs baseline: 1.0237x; 1.0237x over previous
import functools

import jax
import jax.numpy as jnp
from jax import lax
from jax.experimental import pallas as pl
from jax.experimental.pallas import tpu as pltpu
from jax.experimental.pallas import tpu_sc as plsc

DIM = 64
BATCH = 16384
NUM_CORES = 2
NUM_SUBCORES = 16
NUM_WORKERS = NUM_CORES * NUM_SUBCORES
BPW = BATCH // NUM_WORKERS
CHUNK = 128
NCHUNK = BPW // CHUNK
CGROUPS = CHUNK // 16
_PACK_BW = 512


def _pack_half(n):
    return ((n // 2 + _PACK_BW - 1) // _PACK_BW) * _PACK_BW


UHALF = _pack_half(100000)
IHALF = _pack_half(1000000)

_mesh = plsc.VectorSubcoreMesh(core_axis_name="c", subcore_axis_name="s")


@functools.partial(
    pl.kernel,
    out_type=(
        jax.ShapeDtypeStruct((BATCH,), jnp.float32),
        jax.ShapeDtypeStruct((BATCH,), jnp.float32),
    ),
    mesh=_mesh,
    scratch_types=[
        pltpu.VMEM((NCHUNK, CHUNK), jnp.int32),
        pltpu.VMEM((NCHUNK, CHUNK), jnp.int32),
        pltpu.VMEM((NCHUNK, CHUNK), jnp.int32),
        pltpu.VMEM((NCHUNK, CHUNK), jnp.int32),
        pltpu.VMEM((NCHUNK, CHUNK), jnp.int32),
        pltpu.VMEM((NCHUNK, CHUNK), jnp.int32),
        pltpu.VMEM((CHUNK, 2 * DIM), jnp.float32),
        pltpu.VMEM((CHUNK, 2 * DIM), jnp.float32),
        pltpu.VMEM((CHUNK, 2 * DIM), jnp.float32),
        pltpu.VMEM((CHUNK, 2 * DIM), jnp.float32),
        pltpu.VMEM((CHUNK, 2 * DIM), jnp.float32),
        pltpu.VMEM((CHUNK, 2 * DIM), jnp.float32),
        pltpu.VMEM((BPW,), jnp.float32),
        pltpu.VMEM((BPW,), jnp.float32),
        pltpu.SemaphoreType.DMA,
    ],
    compiler_params=pltpu.CompilerParams(
        needs_layout_passes=False, use_tc_tiling_on_sc=True
    ),
)
def _bprmf_sc(u_hbm, pi_hbm, ni_hbm, user_t, item_t, pos_hbm, neg_hbm,
              u_v, pi_v, ni_v, uh_v, ph_v, nh_v,
              ue_v0, ue_v1, pe_v0, pe_v1, ne_v0, ne_v1,
              pos_v, neg_v, sem):
    wid = lax.axis_index("s") * NUM_CORES + lax.axis_index("c")
    lane = lax.iota(jnp.int32, 16)
    ue_b = (ue_v0, ue_v1)
    pe_b = (pe_v0, pe_v1)
    ne_b = (ne_v0, ne_v1)

    pltpu.sync_copy(u_hbm.at[pl.ds(wid * NCHUNK, NCHUNK)], u_v)
    pltpu.sync_copy(pi_hbm.at[pl.ds(wid * NCHUNK, NCHUNK)], pi_v)
    pltpu.sync_copy(ni_hbm.at[pl.ds(wid * NCHUNK, NCHUNK)], ni_v)

    for j in range(NCHUNK):
        for g in range(CGROUPS):
            s = pl.ds(g * 16, 16)
            uv, pv, nv = u_v[j, s], pi_v[j, s], ni_v[j, s]
            uh_v[j, s] = uv - jnp.where(uv >= UHALF, UHALF, 0)
            ph_v[j, s] = pv - jnp.where(pv >= IHALF, IHALF, 0)
            nh_v[j, s] = nv - jnp.where(nv >= IHALF, IHALF, 0)

    def fire(j, buf):
        return (
            pltpu.async_copy(user_t.at[uh_v.at[j]], ue_b[buf], sem),
            pltpu.async_copy(item_t.at[ph_v.at[j]], pe_b[buf], sem),
            pltpu.async_copy(item_t.at[nh_v.at[j]], ne_b[buf], sem),
        )

    def compute(j, buf):
        ue, pe, ne = ue_b[buf], pe_b[buf], ne_b[buf]

        def group_body(g, carry):
            s = pl.ds(g * 16, 16)
            ucol0 = jnp.where(u_v[j, s] >= UHALF, DIM, 0)
            pcol0 = jnp.where(pi_v[j, s] >= IHALF, DIM, 0)
            ncol0 = jnp.where(ni_v[j, s] >= IHALF, DIM, 0)
            rows = g * 16 + lane
            acc_p = jnp.zeros((16,), jnp.float32)
            acc_n = jnp.zeros((16,), jnp.float32)
            for d in range(DIM):
                rot = jnp.bitwise_and(lane + d, DIM - 1)
                a = plsc.load_gather(ue, [rows, ucol0 + rot])
                b = plsc.load_gather(pe, [rows, pcol0 + rot])
                c = plsc.load_gather(ne, [rows, ncol0 + rot])
                acc_p = acc_p + a * b
                acc_n = acc_n + a * c
            pos_v[pl.ds(j * CHUNK + g * 16, 16)] = acc_p
            neg_v[pl.ds(j * CHUNK + g * 16, 16)] = acc_n
            return carry

        lax.fori_loop(0, CGROUPS, group_body, 0)

    inflight = fire(0, 0)
    for j in range(NCHUNK):
        buf = j % 2
        for cp in inflight:
            cp.wait()
        if j + 1 < NCHUNK:
            inflight = fire(j + 1, 1 - buf)
        compute(j, buf)

    pltpu.sync_copy(pos_v, pos_hbm.at[pl.ds(wid * BPW, BPW)])
    pltpu.sync_copy(neg_v, neg_hbm.at[pl.ds(wid * BPW, BPW)])


def _pack_body(lo_ref, hi_ref, eye_ref, out_ref):
    z = jnp.concatenate([lo_ref[...], hi_ref[...]], axis=0)
    out_ref[...] = jax.lax.dot_general(
        z, eye_ref[...],
        dimension_numbers=(((0,), (0,)), ((), ())),
        preferred_element_type=jnp.float32,
    )


def _tc_pack(t_T):
    n = t_T.shape[1]
    half = _pack_half(n)
    half_blocks = half // _PACK_BW
    eye = jnp.eye(2 * DIM, dtype=jnp.float32)
    return pl.pallas_call(
        _pack_body,
        grid=(half_blocks,),
        in_specs=[
            pl.BlockSpec((DIM, _PACK_BW), lambda j: (0, j)),
            pl.BlockSpec((DIM, _PACK_BW), lambda j, hb=half_blocks: (0, j + hb)),
            pl.BlockSpec((2 * DIM, 2 * DIM), lambda j: (0, 0)),
        ],
        out_specs=pl.BlockSpec((_PACK_BW, 2 * DIM), lambda j: (j, 0)),
        out_shape=jax.ShapeDtypeStruct((half, 2 * DIM), jnp.float32),
        compiler_params=pltpu.CompilerParams(fuse_transposed_lhs_in_matmul=True),
    )(t_T, t_T, eye)


def kernel(u, pi, ni, user_emb, item_emb):
    u2 = u.astype(jnp.int32).reshape(BATCH // CHUNK, CHUNK)
    pi2 = pi.astype(jnp.int32).reshape(BATCH // CHUNK, CHUNK)
    ni2 = ni.astype(jnp.int32).reshape(BATCH // CHUNK, CHUNK)
    user2 = _tc_pack(user_emb.T)
    item2 = _tc_pack(item_emb.T)
    return _bprmf_sc(u2, pi2, ni2, user2, item2)


# speedup vs baseline: 1.1845x; 1.1570x over previous
import functools

import jax
import jax.numpy as jnp
from jax import lax
from jax.experimental import pallas as pl
from jax.experimental.pallas import tpu as pltpu
from jax.experimental.pallas import tpu_sc as plsc

DIM = 64
BATCH = 16384
NUM_CORES = 2
NUM_SUBCORES = 16
NUM_WORKERS = NUM_CORES * NUM_SUBCORES
BPW = BATCH // NUM_WORKERS
CHUNK = 128
NCHUNK = BPW // CHUNK
CGROUPS = CHUNK // 16

_mesh = plsc.VectorSubcoreMesh(core_axis_name="c", subcore_axis_name="s")


@functools.partial(
    pl.kernel,
    out_type=(
        jax.ShapeDtypeStruct((BATCH,), jnp.float32),
        jax.ShapeDtypeStruct((BATCH,), jnp.float32),
    ),
    mesh=_mesh,
    scratch_types=[
        pltpu.VMEM((NCHUNK, CHUNK), jnp.int32),
        pltpu.VMEM((NCHUNK, CHUNK), jnp.int32),
        pltpu.VMEM((NCHUNK, CHUNK), jnp.int32),
        pltpu.VMEM((NCHUNK, CHUNK), jnp.int32),
        pltpu.VMEM((NCHUNK, CHUNK), jnp.int32),
        pltpu.VMEM((NCHUNK, CHUNK), jnp.int32),
        pltpu.VMEM((CHUNK, 2 * DIM), jnp.float32),
        pltpu.VMEM((CHUNK, 2 * DIM), jnp.float32),
        pltpu.VMEM((CHUNK, 2 * DIM), jnp.float32),
        pltpu.VMEM((CHUNK, 2 * DIM), jnp.float32),
        pltpu.VMEM((CHUNK, 2 * DIM), jnp.float32),
        pltpu.VMEM((CHUNK, 2 * DIM), jnp.float32),
        pltpu.VMEM((BPW,), jnp.float32),
        pltpu.VMEM((BPW,), jnp.float32),
        pltpu.SemaphoreType.DMA,
    ],
    compiler_params=pltpu.CompilerParams(
        needs_layout_passes=False, use_tc_tiling_on_sc=True
    ),
)
def _bprmf_sc(u_hbm, pi_hbm, ni_hbm, user_t, item_t, pos_hbm, neg_hbm,
              u_v, pi_v, ni_v, uh_v, ph_v, nh_v,
              ue_v0, ue_v1, pe_v0, pe_v1, ne_v0, ne_v1,
              pos_v, neg_v, sem):
    wid = lax.axis_index("s") * NUM_CORES + lax.axis_index("c")
    lane = lax.iota(jnp.int32, 16)
    ue_b = (ue_v0, ue_v1)
    pe_b = (pe_v0, pe_v1)
    ne_b = (ne_v0, ne_v1)

    pltpu.sync_copy(u_hbm.at[pl.ds(wid * NCHUNK, NCHUNK)], u_v)
    pltpu.sync_copy(pi_hbm.at[pl.ds(wid * NCHUNK, NCHUNK)], pi_v)
    pltpu.sync_copy(ni_hbm.at[pl.ds(wid * NCHUNK, NCHUNK)], ni_v)

    for j in range(NCHUNK):
        for g in range(CGROUPS):
            s = pl.ds(g * 16, 16)
            uh_v[j, s] = jnp.right_shift(u_v[j, s], 1)
            ph_v[j, s] = jnp.right_shift(pi_v[j, s], 1)
            nh_v[j, s] = jnp.right_shift(ni_v[j, s], 1)

    def fire(j, buf):
        return (
            pltpu.async_copy(user_t.at[uh_v.at[j]], ue_b[buf], sem),
            pltpu.async_copy(item_t.at[ph_v.at[j]], pe_b[buf], sem),
            pltpu.async_copy(item_t.at[nh_v.at[j]], ne_b[buf], sem),
        )

    def compute(j, buf):
        ue, pe, ne = ue_b[buf], pe_b[buf], ne_b[buf]

        def group_body(g, carry):
            s = pl.ds(g * 16, 16)
            ucol0 = jnp.left_shift(jnp.bitwise_and(u_v[j, s], 1), 6)
            pcol0 = jnp.left_shift(jnp.bitwise_and(pi_v[j, s], 1), 6)
            ncol0 = jnp.left_shift(jnp.bitwise_and(ni_v[j, s], 1), 6)
            rows = g * 16 + lane
            acc_p = jnp.zeros((16,), jnp.float32)
            acc_n = jnp.zeros((16,), jnp.float32)
            for d in range(DIM):
                rot = jnp.bitwise_and(lane + d, DIM - 1)
                a = plsc.load_gather(ue, [rows, ucol0 + rot])
                b = plsc.load_gather(pe, [rows, pcol0 + rot])
                c = plsc.load_gather(ne, [rows, ncol0 + rot])
                acc_p = acc_p + a * b
                acc_n = acc_n + a * c
            pos_v[pl.ds(j * CHUNK + g * 16, 16)] = acc_p
            neg_v[pl.ds(j * CHUNK + g * 16, 16)] = acc_n
            return carry

        lax.fori_loop(0, CGROUPS, group_body, 0)

    inflight = fire(0, 0)
    for j in range(NCHUNK):
        buf = j % 2
        for cp in inflight:
            cp.wait()
        if j + 1 < NCHUNK:
            inflight = fire(j + 1, 1 - buf)
        compute(j, buf)

    pltpu.sync_copy(pos_v, pos_hbm.at[pl.ds(wid * BPW, BPW)])
    pltpu.sync_copy(neg_v, neg_hbm.at[pl.ds(wid * BPW, BPW)])


def kernel(u, pi, ni, user_emb, item_emb):
    u2 = u.astype(jnp.int32).reshape(BATCH // CHUNK, CHUNK)
    pi2 = pi.astype(jnp.int32).reshape(BATCH // CHUNK, CHUNK)
    ni2 = ni.astype(jnp.int32).reshape(BATCH // CHUNK, CHUNK)
    user2 = user_emb.reshape(-1, 2 * DIM)
    item2 = item_emb.reshape(-1, 2 * DIM)
    return _bprmf_sc(u2, pi2, ni2, user2, item2)
